# baseline (speedup 1.0000x reference)
import jax
import jax.numpy as jnp
from jax import lax
from jax.experimental import pallas as pl
from jax.experimental.pallas import tpu as pltpu
from jax.experimental.pallas import tpu_sc as plsc

N = 10000
E = 320000
F_IN = 127
H = 128
C = 63

NTILES = 32
EPT = E // NTILES
K = 80
NCH = EPT // K
RPS = 624
TAIL = N - 16 * RPS

BN = 2000


def _pre_body(x_ref, wmlp_ref, bmlp_ref, w12_ref, a4_ref, t1_ref, t2_ref,
              s4_ref, x3_ref):
    x0 = jax.nn.relu(
        jnp.dot(x_ref[...], wmlp_ref[...], preferred_element_type=jnp.float32)
        + bmlp_ref[...]
    )
    h12 = jnp.dot(x0, w12_ref[...], preferred_element_type=jnp.float32)
    ones = jnp.ones((x0.shape[0], 1), jnp.float32)
    t1_ref[...] = jnp.concatenate([h12[:, :C], ones], axis=1)
    t2_ref[...] = jnp.concatenate([h12[:, C:], ones], axis=1)
    s4_ref[...] = jnp.dot(h12, a4_ref[...], preferred_element_type=jnp.float32)
    x3_ref[...] = jnp.max(x0, axis=1, keepdims=True)


def _pre_call(x, wmlp, bmlp, w12, a4):
    grid = (N // BN,)
    return pl.pallas_call(
        _pre_body,
        grid=grid,
        in_specs=[
            pl.BlockSpec((BN, F_IN), lambda i: (i, 0)),
            pl.BlockSpec((F_IN, H), lambda i: (0, 0)),
            pl.BlockSpec((1, H), lambda i: (0, 0)),
            pl.BlockSpec((H, 2 * C), lambda i: (0, 0)),
            pl.BlockSpec((2 * C, 4), lambda i: (0, 0)),
        ],
        out_specs=[
            pl.BlockSpec((BN, C + 1), lambda i: (i, 0)),
            pl.BlockSpec((BN, C + 1), lambda i: (i, 0)),
            pl.BlockSpec((BN, 4), lambda i: (i, 0)),
            pl.BlockSpec((BN, 1), lambda i: (i, 0)),
        ],
        out_shape=[
            jax.ShapeDtypeStruct((N, C + 1), jnp.float32),
            jax.ShapeDtypeStruct((N, C + 1), jnp.float32),
            jax.ShapeDtypeStruct((N, 4), jnp.float32),
            jax.ShapeDtypeStruct((N, 1), jnp.float32),
        ],
    )(x, wmlp, bmlp, w12, a4)


def _leaky(v):
    return jnp.where(v > 0, v, 0.2 * v)


def _lane_bcast(vec, lane):
    idx = jnp.full((16, 1), lane, jnp.int32)
    dnums = lax.GatherDimensionNumbers(
        offset_dims=(), collapsed_slice_dims=(0,), start_index_map=(0,))
    return lax.gather(vec, idx, dnums, (1,),
                      mode=lax.GatherScatterMode.PROMISE_IN_BOUNDS)


D = C + 1


def _sc_body(t1_hbm, t2_hbm, s_hbm, src_hbm, dst_hbm, z_hbm, acc_hbm,
             s1s_v, s1d_v, s2s_v, s2d_v, src_v, dst_v, wbuf,
             rbuf0, rbuf1, gsem0, gsem1, ssem0, ssem1, acc):
    cid = lax.axis_index("c")
    sid = lax.axis_index("s")
    wid = cid * 16 + sid
    rbufs = (rbuf0, rbuf1)
    gsems = (gsem0, gsem1)
    ssems = (ssem0, ssem1)
    aoff = pl.multiple_of(sid * RPS, 16)

    pltpu.sync_copy(s_hbm.at[0], s1s_v)
    pltpu.sync_copy(s_hbm.at[1], s1d_v)
    pltpu.sync_copy(s_hbm.at[2], s2s_v)
    pltpu.sync_copy(s_hbm.at[3], s2d_v)
    pltpu.sync_copy(src_hbm.at[wid], src_v)
    pltpu.sync_copy(dst_hbm.at[wid], dst_v)

    def run_phase(l):
        t_hbm = (t1_hbm, t2_hbm)[l]
        ss_v = (s1s_v, s2s_v)[l]
        sd_v = (s1d_v, s2d_v)[l]

        pltpu.sync_copy(z_hbm, acc.at[pl.ds(aoff, RPS)])

        @pl.when(sid == 0)
        def _():
            pltpu.sync_copy(z_hbm.at[pl.ds(0, TAIL)],
                            acc.at[pl.ds(16 * RPS, TAIL)])

        plsc.subcore_barrier()

        def compute_w(j, slot):
            srow = src_v.at[j]
            drow = dst_v.at[j]
            for i in range(K // 16):
                s16 = srow[pl.ds(16 * i, 16)]
                d16 = drow[pl.ds(16 * i, 16)]
                a = plsc.load_gather(ss_v, [s16]) + plsc.load_gather(sd_v, [d16])
                wbuf[slot, pl.ds(16 * i, 16)] = jnp.exp(_leaky(a))

        def scale(b):
            rb = rbufs[b]

            @pl.loop(0, K // 16, unroll=1)
            def _(gg):
                wv = wbuf[b, pl.ds(gg * 16, 16)]
                base = pl.multiple_of(gg * 16, 16)
                for rr in range(16):
                    wr = _lane_bcast(wv, rr)
                    r = base + rr
                    for q in range(D // 16):
                        rb[r, pl.ds(q * 16, 16)] = rb[r, pl.ds(q * 16, 16)] * wr

        def fire_gather(j, b):
            pltpu.async_copy(t_hbm.at[src_v.at[j]], rbufs[b], gsems[b])

        def wait_gather(j, b):
            pltpu.make_async_copy(t_hbm.at[src_v.at[j]], rbufs[b],
                                  gsems[b]).wait()

        def fire_scatter(j, b):
            pltpu.async_copy(rbufs[b], acc.at[dst_v.at[j]], ssems[b], add=True)

        def wait_scatter(j, b):
            pltpu.make_async_copy(rbufs[b], acc.at[dst_v.at[j]],
                                  ssems[b]).wait()

        compute_w(0, 0)
        fire_gather(0, 0)
        wait_gather(0, 0)
        scale(0)
        fire_scatter(0, 0)
        compute_w(1, 1)
        fire_gather(1, 1)

        @pl.loop(1, NCH - 2, step=2)
        def _(jo):
            for db, off in ((1, 0), (0, 1)):
                j = jo + off
                wait_gather(j, db)
                scale(db)
                fire_scatter(j, db)
                nb = 1 - db
                compute_w(j + 1, nb)
                wait_scatter(j - 1, nb)
                fire_gather(j + 1, nb)

        j = NCH - 2
        wait_gather(j, 1)
        scale(1)
        fire_scatter(j, 1)
        compute_w(j + 1, 0)
        wait_scatter(j - 1, 0)
        fire_gather(j + 1, 0)
        wait_gather(j + 1, 0)
        scale(0)
        fire_scatter(j + 1, 0)
        wait_scatter(j, 1)
        wait_scatter(j + 1, 0)

        plsc.subcore_barrier()
        pltpu.sync_copy(
            acc.at[pl.ds(aoff, RPS)],
            acc_hbm.at[l, cid, pl.ds(aoff, RPS)],
        )

        @pl.when(sid == 0)
        def _():
            pltpu.sync_copy(
                acc.at[pl.ds(16 * RPS, TAIL)],
                acc_hbm.at[l, cid, pl.ds(16 * RPS, TAIL)],
            )

    run_phase(0)
    run_phase(1)


def _sc_call(t1, t2, s4, srcg, dstg, z):
    mesh = plsc.VectorSubcoreMesh(core_axis_name="c", subcore_axis_name="s")
    fn = pl.kernel(
        _sc_body,
        out_type=jax.ShapeDtypeStruct((2, 2, N, D), jnp.float32),
        mesh=mesh,
        compiler_params=pltpu.CompilerParams(
            needs_layout_passes=False, use_tc_tiling_on_sc=False),
        scratch_types=[
            pltpu.VMEM((N,), jnp.float32),
            pltpu.VMEM((N,), jnp.float32),
            pltpu.VMEM((N,), jnp.float32),
            pltpu.VMEM((N,), jnp.float32),
            pltpu.VMEM((NCH, K), jnp.int32),
            pltpu.VMEM((NCH, K), jnp.int32),
            pltpu.VMEM((2, K), jnp.float32),
            pltpu.VMEM((K, D), jnp.float32),
            pltpu.VMEM((K, D), jnp.float32),
            pltpu.SemaphoreType.DMA,
            pltpu.SemaphoreType.DMA,
            pltpu.SemaphoreType.DMA,
            pltpu.SemaphoreType.DMA,
            pltpu.VMEM_SHARED((N, D), jnp.float32),
        ],
    )
    return fn(t1, t2, s4, srcg, dstg, z)


def _post_body(acc_ref, x_ref, x3_ref, g_ref, o_ref):
    a0 = acc_ref[0, 0] + acc_ref[0, 1]
    a1 = acc_ref[1, 0] + acc_ref[1, 1]
    g = g_ref[...]

    def ln(nmr, dnm, gamma, beta):
        o = nmr / (dnm + 1e-16)
        mu = jnp.mean(o, axis=1, keepdims=True)
        var = jnp.mean((o - mu) ** 2, axis=1, keepdims=True)
        return (o - mu) * lax.rsqrt(var + 1e-5) * gamma + beta

    x1 = jax.nn.relu(ln(a0[:, :C], a0[:, C:], g[0], g[1]))
    x2 = ln(a1[:, :C], a1[:, C:], g[2], g[3])
    lng = jnp.concatenate([x1, x2, x3_ref[...]], axis=1) + x_ref[...]
    m = jnp.max(lng, axis=1, keepdims=True)
    ex = jnp.exp(lng - m)
    o_ref[...] = lng - m - jnp.log(jnp.sum(ex, axis=1, keepdims=True))


def _post_call(acc, x, x3, g):
    grid = (N // BN,)
    return pl.pallas_call(
        _post_body,
        grid=grid,
        in_specs=[
            pl.BlockSpec((2, 2, BN, D), lambda i: (0, 0, i, 0)),
            pl.BlockSpec((BN, F_IN), lambda i: (i, 0)),
            pl.BlockSpec((BN, 1), lambda i: (i, 0)),
            pl.BlockSpec((4, C), lambda i: (0, 0)),
        ],
        out_specs=pl.BlockSpec((BN, F_IN), lambda i: (i, 0)),
        out_shape=jax.ShapeDtypeStruct((N, F_IN), jnp.float32),
    )(acc, x, x3, g)


def kernel(x, edge_index, W_mlp, b_mlp, W1, a1_src, a1_dst, g1, be1,
           W2, a2_src, a2_dst, g2, be2):
    w12 = jnp.concatenate([W1, W2], axis=1)
    a4 = jnp.zeros((2 * C, 4), jnp.float32)
    a4 = a4.at[:C, 0].set(a1_src).at[:C, 1].set(a1_dst)
    a4 = a4.at[C:, 2].set(a2_src).at[C:, 3].set(a2_dst)
    t1, t2, s4, x3 = _pre_call(x, W_mlp, b_mlp.reshape(1, H), w12, a4)
    s4 = s4.T
    srcg = edge_index[0].reshape(NTILES, NCH, K)
    dstg = edge_index[1].reshape(NTILES, NCH, K)
    z = jnp.zeros((RPS, D), jnp.float32)
    acc = _sc_call(t1, t2, s4, srcg, dstg, z)
    g = jnp.stack([g1, be1, g2, be2])
    return _post_call(acc, x, x3, g)


# speedup vs baseline: 2.2035x; 2.2035x over previous
import jax
import jax.numpy as jnp
from jax import lax
from jax.experimental import pallas as pl
from jax.experimental.pallas import tpu as pltpu
from jax.experimental.pallas import tpu_sc as plsc

N = 10000
E = 320000
F_IN = 127
H = 128
C = 63

NTILES = 32
EPT = E // NTILES
K = 80
NCH = EPT // K
RPS = 624
TAIL = N - 16 * RPS

BN = 2000


def _pre_body(x_ref, wmlp_ref, bmlp_ref, w12_ref, a4_ref, t1_ref, t2_ref,
              s4_ref, x3_ref):
    x0 = jax.nn.relu(
        jnp.dot(x_ref[...], wmlp_ref[...], preferred_element_type=jnp.float32)
        + bmlp_ref[...]
    )
    h12 = jnp.dot(x0, w12_ref[...], preferred_element_type=jnp.float32)
    ones = jnp.ones((x0.shape[0], 1), jnp.float32)
    t1_ref[...] = jnp.concatenate([h12[:, :C], ones], axis=1)
    t2_ref[...] = jnp.concatenate([h12[:, C:], ones], axis=1)
    s4_ref[...] = jnp.dot(h12, a4_ref[...], preferred_element_type=jnp.float32)
    x3_ref[...] = jnp.max(x0, axis=1, keepdims=True)


def _pre_call(x, wmlp, bmlp, w12, a4):
    grid = (N // BN,)
    return pl.pallas_call(
        _pre_body,
        grid=grid,
        in_specs=[
            pl.BlockSpec((BN, F_IN), lambda i: (i, 0)),
            pl.BlockSpec((F_IN, H), lambda i: (0, 0)),
            pl.BlockSpec((1, H), lambda i: (0, 0)),
            pl.BlockSpec((H, 2 * C), lambda i: (0, 0)),
            pl.BlockSpec((2 * C, 4), lambda i: (0, 0)),
        ],
        out_specs=[
            pl.BlockSpec((BN, C + 1), lambda i: (i, 0)),
            pl.BlockSpec((BN, C + 1), lambda i: (i, 0)),
            pl.BlockSpec((BN, 4), lambda i: (i, 0)),
            pl.BlockSpec((BN, 1), lambda i: (i, 0)),
        ],
        out_shape=[
            jax.ShapeDtypeStruct((N, C + 1), jnp.float32),
            jax.ShapeDtypeStruct((N, C + 1), jnp.float32),
            jax.ShapeDtypeStruct((N, 4), jnp.float32),
            jax.ShapeDtypeStruct((N, 1), jnp.float32),
        ],
    )(x, wmlp, bmlp, w12, a4)


def _leaky(v):
    return jnp.where(v > 0, v, 0.2 * v)


D = C + 1


def _sc_body(t1_hbm, t2_hbm, s_hbm, src_hbm, dst_hbm, z_hbm, acc_hbm,
             s1s_v, s1d_v, s2s_v, s2d_v, src_v, dst_v, wbuf,
             rbuf0, rbuf1, rbuf2, gsem0, gsem1, gsem2,
             ssem0, ssem1, ssem2, acc):
    cid = lax.axis_index("c")
    sid = lax.axis_index("s")
    wid = cid * 16 + sid
    rbufs = (rbuf0, rbuf1, rbuf2)
    gsems = (gsem0, gsem1, gsem2)
    ssems = (ssem0, ssem1, ssem2)
    aoff = pl.multiple_of(sid * RPS, 16)

    pltpu.sync_copy(s_hbm.at[0], s1s_v)
    pltpu.sync_copy(s_hbm.at[1], s1d_v)
    pltpu.sync_copy(s_hbm.at[2], s2s_v)
    pltpu.sync_copy(s_hbm.at[3], s2d_v)
    pltpu.sync_copy(src_hbm.at[wid], src_v)
    pltpu.sync_copy(dst_hbm.at[wid], dst_v)

    def run_phase(l):
        t_hbm = (t1_hbm, t2_hbm)[l]
        ss_v = (s1s_v, s2s_v)[l]
        sd_v = (s1d_v, s2d_v)[l]

        pltpu.sync_copy(z_hbm, acc.at[pl.ds(aoff, RPS)])

        @pl.when(sid == 0)
        def _():
            pltpu.sync_copy(z_hbm.at[pl.ds(0, TAIL)],
                            acc.at[pl.ds(16 * RPS, TAIL)])

        plsc.subcore_barrier()

        def compute_w(j, slot):
            srow = src_v.at[j]
            drow = dst_v.at[j]
            for i in range(K // 16):
                s16 = srow[pl.ds(16 * i, 16)]
                d16 = drow[pl.ds(16 * i, 16)]
                a = plsc.load_gather(ss_v, [s16]) + plsc.load_gather(sd_v, [d16])
                wbuf[slot, pl.ds(16 * i, 16)] = jnp.exp(_leaky(a))

        def scale(b):
            rb = rbufs[b]

            @pl.loop(0, K, unroll=4)
            def _(r):
                ridx = jnp.full((16,), r, jnp.int32)
                wv = plsc.load_gather(wbuf.at[b], [ridx])
                for q in range(D // 16):
                    rb[r, pl.ds(q * 16, 16)] = rb[r, pl.ds(q * 16, 16)] * wv

        def fire_gather(j, b):
            pltpu.async_copy(t_hbm.at[src_v.at[j]], rbufs[b], gsems[b])

        def wait_gather(j, b):
            pltpu.make_async_copy(t_hbm.at[src_v.at[j]], rbufs[b],
                                  gsems[b]).wait()

        def fire_scatter(j, b):
            pltpu.async_copy(rbufs[b], acc.at[dst_v.at[j]], ssems[b], add=True)

        def wait_scatter(j, b):
            pltpu.make_async_copy(rbufs[b], acc.at[dst_v.at[j]],
                                  ssems[b]).wait()

        def chunk_body(j, b, pb, fire_next):
            compute_w(j, b)
            wait_gather(j, b)
            scale(b)
            fire_scatter(j, b)
            if fire_next:
                wait_scatter(j - 1, pb)
                fire_gather(j + 2, pb)

        fire_gather(0, 0)
        fire_gather(1, 1)
        compute_w(0, 0)
        wait_gather(0, 0)
        scale(0)
        fire_scatter(0, 0)
        fire_gather(2, 2)

        @pl.loop(1, NCH - 4, step=3)
        def _(jo):
            for off in range(3):
                chunk_body(jo + off, (1 + off) % 3, off % 3, True)

        chunk_body(NCH - 4, (NCH - 4) % 3, (NCH - 2) % 3, True)
        chunk_body(NCH - 3, (NCH - 3) % 3, (NCH - 1) % 3, True)
        chunk_body(NCH - 2, (NCH - 2) % 3, 0, False)
        chunk_body(NCH - 1, (NCH - 1) % 3, 0, False)
        wait_scatter(NCH - 3, (NCH - 3) % 3)
        wait_scatter(NCH - 2, (NCH - 2) % 3)
        wait_scatter(NCH - 1, (NCH - 1) % 3)

        plsc.subcore_barrier()
        pltpu.sync_copy(
            acc.at[pl.ds(aoff, RPS)],
            acc_hbm.at[l, cid, pl.ds(aoff, RPS)],
        )

        @pl.when(sid == 0)
        def _():
            pltpu.sync_copy(
                acc.at[pl.ds(16 * RPS, TAIL)],
                acc_hbm.at[l, cid, pl.ds(16 * RPS, TAIL)],
            )

    run_phase(0)
    run_phase(1)


def _sc_call(t1, t2, s4, srcg, dstg, z):
    mesh = plsc.VectorSubcoreMesh(core_axis_name="c", subcore_axis_name="s")
    fn = pl.kernel(
        _sc_body,
        out_type=jax.ShapeDtypeStruct((2, 2, N, D), jnp.float32),
        mesh=mesh,
        compiler_params=pltpu.CompilerParams(
            needs_layout_passes=False, use_tc_tiling_on_sc=False),
        scratch_types=[
            pltpu.VMEM((N,), jnp.float32),
            pltpu.VMEM((N,), jnp.float32),
            pltpu.VMEM((N,), jnp.float32),
            pltpu.VMEM((N,), jnp.float32),
            pltpu.VMEM((NCH, K), jnp.int32),
            pltpu.VMEM((NCH, K), jnp.int32),
            pltpu.VMEM((3, K), jnp.float32),
            pltpu.VMEM((K, D), jnp.float32),
            pltpu.VMEM((K, D), jnp.float32),
            pltpu.VMEM((K, D), jnp.float32),
            pltpu.SemaphoreType.DMA,
            pltpu.SemaphoreType.DMA,
            pltpu.SemaphoreType.DMA,
            pltpu.SemaphoreType.DMA,
            pltpu.SemaphoreType.DMA,
            pltpu.SemaphoreType.DMA,
            pltpu.VMEM_SHARED((N, D), jnp.float32),
        ],
    )
    return fn(t1, t2, s4, srcg, dstg, z)


def _post_body(acc_ref, x_ref, x3_ref, g_ref, o_ref):
    a0 = acc_ref[0, 0] + acc_ref[0, 1]
    a1 = acc_ref[1, 0] + acc_ref[1, 1]
    g = g_ref[...]

    def ln(nmr, dnm, gamma, beta):
        o = nmr / (dnm + 1e-16)
        mu = jnp.mean(o, axis=1, keepdims=True)
        var = jnp.mean((o - mu) ** 2, axis=1, keepdims=True)
        return (o - mu) * lax.rsqrt(var + 1e-5) * gamma + beta

    x1 = jax.nn.relu(ln(a0[:, :C], a0[:, C:], g[0], g[1]))
    x2 = ln(a1[:, :C], a1[:, C:], g[2], g[3])
    lng = jnp.concatenate([x1, x2, x3_ref[...]], axis=1) + x_ref[...]
    m = jnp.max(lng, axis=1, keepdims=True)
    ex = jnp.exp(lng - m)
    o_ref[...] = lng - m - jnp.log(jnp.sum(ex, axis=1, keepdims=True))


def _post_call(acc, x, x3, g):
    grid = (N // BN,)
    return pl.pallas_call(
        _post_body,
        grid=grid,
        in_specs=[
            pl.BlockSpec((2, 2, BN, D), lambda i: (0, 0, i, 0)),
            pl.BlockSpec((BN, F_IN), lambda i: (i, 0)),
            pl.BlockSpec((BN, 1), lambda i: (i, 0)),
            pl.BlockSpec((4, C), lambda i: (0, 0)),
        ],
        out_specs=pl.BlockSpec((BN, F_IN), lambda i: (i, 0)),
        out_shape=jax.ShapeDtypeStruct((N, F_IN), jnp.float32),
    )(acc, x, x3, g)


def kernel(x, edge_index, W_mlp, b_mlp, W1, a1_src, a1_dst, g1, be1,
           W2, a2_src, a2_dst, g2, be2):
    w12 = jnp.concatenate([W1, W2], axis=1)
    a4 = jnp.zeros((2 * C, 4), jnp.float32)
    a4 = a4.at[:C, 0].set(a1_src).at[:C, 1].set(a1_dst)
    a4 = a4.at[C:, 2].set(a2_src).at[C:, 3].set(a2_dst)
    t1, t2, s4, x3 = _pre_call(x, W_mlp, b_mlp.reshape(1, H), w12, a4)
    s4 = s4.T
    srcg = edge_index[0].reshape(NTILES, NCH, K)
    dstg = edge_index[1].reshape(NTILES, NCH, K)
    z = jnp.zeros((RPS, D), jnp.float32)
    acc = _sc_call(t1, t2, s4, srcg, dstg, z)
    g = jnp.stack([g1, be1, g2, be2])
    return _post_call(acc, x, x3, g)
